# baseline (speedup 1.0000x reference)
import functools

import jax
import jax.numpy as jnp
from jax import lax
from jax.experimental import pallas as pl
from jax.experimental.pallas import tpu as pltpu
from jax.experimental.pallas import tpu_sc as plsc

N = 10000
NP = 10240
E = 320000
D = 128
HALF = 64
K = 3

NC = 2
NS = 16
LANES = 16

CHUNK = 96
NCH = 210
EPT = NCH * CHUNK
EP = EPT * NS
EPC = EP // CHUNK
RPT = NP // NS
ZR = 64
NBUF = 6
ELA = 4
GLA = 2


def _sc_body(x2, pack, t1, t2, t3,
             tab_a, tab_b, ebuf, g, zbuf, sem_e, sem_g, sem_s):
    c = lax.axis_index("c")
    s = lax.axis_index("s")
    coff = c * NP
    rbase = s * RPT
    cbase = s * NCH

    pltpu.sync_copy(x2.at[pl.ds(coff + rbase, RPT)],
                    tab_a.at[pl.ds(rbase, RPT)])

    def z_body(r, carry):
        for j in range(HALF // LANES):
            zbuf[r, pl.ds(j * LANES, LANES)] = jnp.zeros((LANES,), jnp.float32)
        return carry

    lax.fori_loop(0, ZR, z_body, 0)
    for j in range(RPT // ZR):
        pltpu.sync_copy(zbuf, tab_b.at[pl.ds(rbase + j * ZR, ZR)])
    plsc.subcore_barrier()

    def eload_issue(ch, b):
        pltpu.async_copy(pack.at[cbase + ch], ebuf.at[b], sem_e[b])

    def eload_wait(b):
        pltpu.make_async_copy(pack.at[cbase], ebuf.at[b], sem_e[b]).wait()

    def spmv_round(src_tab, dst_tab, vrow, out_hbm):
        def gather_issue(ch, b):
            pltpu.async_copy(src_tab.at[ebuf.at[b, 0]], g.at[b], sem_g[b])

        def gather_wait(b):
            pltpu.make_async_copy(src_tab.at[ebuf.at[b, 0]], g.at[b],
                                  sem_g[b]).wait()

        def scatter_issue(b):
            pltpu.async_copy(g.at[b], dst_tab.at[ebuf.at[b, 1]], sem_s[b],
                             add=True)

        def scatter_wait(b):
            pltpu.make_async_copy(g.at[b], dst_tab.at[ebuf.at[b, 1]],
                                  sem_s[b]).wait()

        def scale(b):
            def e_body(eg, carry):
                vv = plsc.bitcast(ebuf[b, vrow, pl.ds(eg * LANES, LANES)],
                                  jnp.float32)
                for u0 in range(0, LANES, 4):
                    vals4 = [vv[u0 + u] for u in range(4)]
                    rows4 = [[g[b, eg * LANES + u0 + u, pl.ds(j * LANES, LANES)]
                              for j in range(HALF // LANES)] for u in range(4)]
                    for u in range(4):
                        for j in range(HALF // LANES):
                            sl = pl.ds(j * LANES, LANES)
                            g[b, eg * LANES + u0 + u, sl] = rows4[u][j] * vals4[u]
                return carry

            lax.fori_loop(0, CHUNK // LANES, e_body, 0)

        def compute_stage(ch, b):
            gather_wait(b)
            scale(b)
            scatter_issue(b)

        def iteration(ch, b, swait, do_eload, do_gather):
            if swait:
                scatter_wait((b + ELA) % NBUF)
            if do_eload:
                eload_issue(ch + ELA, (b + ELA) % NBUF)
            if do_gather:
                eload_wait((b + GLA) % NBUF)
                gather_issue(ch + GLA, (b + GLA) % NBUF)
            compute_stage(ch, b)

        for ch in range(ELA):
            eload_issue(ch, ch)
        for ch in range(GLA):
            eload_wait(ch)
            gather_issue(ch, ch)

        for ch in range(NBUF - ELA):
            iteration(ch, ch % NBUF, False, True, True)

        def group_body(m, carry):
            ch0 = (NBUF - ELA) + m * NBUF
            for pos in range(NBUF):
                b = (NBUF - ELA + pos) % NBUF
                iteration(ch0 + pos, b, True, True, True)
            return carry

        lax.fori_loop(0, (NCH - NBUF) // NBUF, group_body, 0)

        for ch in range(NCH - ELA, NCH):
            iteration(ch, ch % NBUF, True, False, ch + GLA < NCH)
        scatter_wait((NCH - 2) % NBUF)
        scatter_wait((NCH - 1) % NBUF)
        plsc.subcore_barrier()

        pltpu.sync_copy(dst_tab.at[pl.ds(rbase, RPT)],
                        out_hbm.at[pl.ds(coff + rbase, RPT)])

    spmv_round(tab_a, tab_b, 2, t1)
    spmv_round(tab_b, tab_a, 3, t2)
    spmv_round(tab_a, tab_b, 4, t3)


_sc_spmv = functools.partial(
    pl.kernel,
    mesh=plsc.VectorSubcoreMesh(core_axis_name="c", subcore_axis_name="s"),
    out_type=[jax.ShapeDtypeStruct((NC * NP, HALF), jnp.float32)] * K,
    scratch_types=[
        pltpu.VMEM_SHARED((NP, HALF), jnp.float32),
        pltpu.VMEM_SHARED((NP, HALF), jnp.float32),
        pltpu.VMEM((NBUF, 5, CHUNK), jnp.int32),
        pltpu.VMEM((NBUF, CHUNK, HALF), jnp.float32),
        pltpu.VMEM((ZR, HALF), jnp.float32),
        [pltpu.SemaphoreType.DMA] * NBUF,
        [pltpu.SemaphoreType.DMA] * NBUF,
        [pltpu.SemaphoreType.DMA] * NBUF,
    ],
    compiler_params=pltpu.CompilerParams(use_tc_tiling_on_sc=False,
                                         needs_layout_passes=False),
)(_sc_body)


BR = 1000
SGN = (1.0, -1.0, -1.0, 1.0)


def _tc_body(x_ref, t1_ref, t2_ref, t3_ref, th_ref, o_ref):
    acc = jnp.dot(x_ref[...], th_ref[0], preferred_element_type=jnp.float32)
    for k, tr in enumerate((t1_ref, t2_ref, t3_ref)):
        tcat = jnp.concatenate([tr[0], tr[1]], axis=1)
        prod = jnp.dot(tcat, th_ref[k + 1], preferred_element_type=jnp.float32)
        acc = acc + SGN[k + 1] * prod
    o_ref[...] = acc


def _tc_proj(x, t1, t2, t3, theta):
    tspec = pl.BlockSpec((2, BR, HALF), lambda i: (0, i, 0))
    return pl.pallas_call(
        _tc_body,
        grid=(N // BR,),
        in_specs=[
            pl.BlockSpec((BR, D), lambda i: (i, 0)),
            tspec, tspec, tspec,
            pl.BlockSpec((K + 1, D, D), lambda i: (0, 0, 0)),
        ],
        out_specs=pl.BlockSpec((BR, D), lambda i: (i, 0)),
        out_shape=jax.ShapeDtypeStruct((N, D), jnp.float32),
    )(x, t1, t2, t3, theta)


def kernel(x, edge_index, edge_vals, theta):
    rows = edge_index[0]
    cols = edge_index[1]
    pad = EP - E
    cols2d = jnp.pad(cols, (0, pad)).reshape(EPC, CHUNK)
    rows2d = jnp.pad(rows, (0, pad)).reshape(EPC, CHUNK)
    vp = jnp.pad(edge_vals, (0, pad))

    def asi32(a):
        return jax.lax.bitcast_convert_type(a, jnp.int32).reshape(EPC, CHUNK)

    pack = jnp.stack(
        [cols2d, rows2d, asi32(-vp), asi32(2.0 * vp), asi32(-2.0 * vp)],
        axis=1)
    rpad = NP - N
    x2 = jnp.concatenate([jnp.pad(x[:, :HALF], ((0, rpad), (0, 0))),
                          jnp.pad(x[:, HALF:], ((0, rpad), (0, 0)))], axis=0)
    t1, t2, t3 = _sc_spmv(x2, pack)
    return _tc_proj(x,
                    t1.reshape(NC, NP, HALF),
                    t2.reshape(NC, NP, HALF),
                    t3.reshape(NC, NP, HALF),
                    theta)


# speedup vs baseline: 1.0722x; 1.0722x over previous
import functools

import jax
import jax.numpy as jnp
from jax import lax
from jax.experimental import pallas as pl
from jax.experimental.pallas import tpu as pltpu
from jax.experimental.pallas import tpu_sc as plsc

N = 10000
NP = 10240
E = 320000
D = 128
HALF = 64
K = 3

NC = 2
NS = 16
LANES = 16

CHUNK = 96
NCH = 210
EPT = NCH * CHUNK
EP = EPT * NS
EPC = EP // CHUNK
RPT = NP // NS
ZR = 64
NBUF = 6
ELA = 4
GLA = 2


def _sc_body(x2, pack, t1, t2, t3,
             tab_a, tab_b, ebuf, g, zbuf, sem_e, sem_g, sem_s):
    c = lax.axis_index("c")
    s = lax.axis_index("s")
    coff = c * NP
    rbase = s * RPT
    cbase = s * NCH

    pltpu.sync_copy(x2.at[pl.ds(coff + rbase, RPT)],
                    tab_a.at[pl.ds(rbase, RPT)])

    def z_body(r, carry):
        for j in range(HALF // LANES):
            zbuf[r, pl.ds(j * LANES, LANES)] = jnp.zeros((LANES,), jnp.float32)
        return carry

    lax.fori_loop(0, ZR, z_body, 0)
    for j in range(RPT // ZR):
        pltpu.sync_copy(zbuf, tab_b.at[pl.ds(rbase + j * ZR, ZR)])
    plsc.subcore_barrier()

    def eload_issue(ch, b):
        pltpu.async_copy(pack.at[cbase + ch], ebuf.at[b], sem_e[b])

    def eload_wait(b):
        pltpu.make_async_copy(pack.at[cbase], ebuf.at[b], sem_e[b]).wait()

    def spmv_round(src_tab, dst_tab, vconst, out_hbm):
        def gather_issue(ch, b):
            pltpu.async_copy(src_tab.at[ebuf.at[b, 0]], g.at[b], sem_g[b])

        def gather_wait(b):
            pltpu.make_async_copy(src_tab.at[ebuf.at[b, 0]], g.at[b],
                                  sem_g[b]).wait()

        def scatter_issue(b):
            pltpu.async_copy(g.at[b], dst_tab.at[ebuf.at[b, 1]], sem_s[b],
                             add=True)

        def scatter_wait(b):
            pltpu.make_async_copy(g.at[b], dst_tab.at[ebuf.at[b, 1]],
                                  sem_s[b]).wait()

        def scale(b):
            def e_body(eg, carry):
                vv = vconst * plsc.bitcast(
                    ebuf[b, 2, pl.ds(eg * LANES, LANES)], jnp.float32)
                for u0 in range(0, LANES, 4):
                    vals4 = [vv[u0 + u] for u in range(4)]
                    rows4 = [[g[b, eg * LANES + u0 + u, pl.ds(j * LANES, LANES)]
                              for j in range(HALF // LANES)] for u in range(4)]
                    for u in range(4):
                        for j in range(HALF // LANES):
                            sl = pl.ds(j * LANES, LANES)
                            g[b, eg * LANES + u0 + u, sl] = rows4[u][j] * vals4[u]
                return carry

            lax.fori_loop(0, CHUNK // LANES, e_body, 0)

        def compute_stage(ch, b):
            gather_wait(b)
            scale(b)
            scatter_issue(b)

        def iteration(ch, b, swait, do_eload, do_gather):
            if swait:
                scatter_wait((b + ELA) % NBUF)
            if do_eload:
                eload_issue(ch + ELA, (b + ELA) % NBUF)
            if do_gather:
                eload_wait((b + GLA) % NBUF)
                gather_issue(ch + GLA, (b + GLA) % NBUF)
            compute_stage(ch, b)

        for ch in range(ELA):
            eload_issue(ch, ch)
        for ch in range(GLA):
            eload_wait(ch)
            gather_issue(ch, ch)

        for ch in range(NBUF - ELA):
            iteration(ch, ch % NBUF, False, True, True)

        def group_body(m, carry):
            ch0 = (NBUF - ELA) + m * NBUF
            for pos in range(NBUF):
                b = (NBUF - ELA + pos) % NBUF
                iteration(ch0 + pos, b, True, True, True)
            return carry

        lax.fori_loop(0, (NCH - NBUF) // NBUF, group_body, 0)

        for ch in range(NCH - ELA, NCH):
            iteration(ch, ch % NBUF, True, False, ch + GLA < NCH)
        scatter_wait((NCH - 2) % NBUF)
        scatter_wait((NCH - 1) % NBUF)
        plsc.subcore_barrier()

        pltpu.sync_copy(dst_tab.at[pl.ds(rbase, RPT)],
                        out_hbm.at[pl.ds(coff + rbase, RPT)])

    spmv_round(tab_a, tab_b, -1.0, t1)
    spmv_round(tab_b, tab_a, 2.0, t2)
    spmv_round(tab_a, tab_b, -2.0, t3)


_sc_spmv = functools.partial(
    pl.kernel,
    mesh=plsc.VectorSubcoreMesh(core_axis_name="c", subcore_axis_name="s"),
    out_type=[jax.ShapeDtypeStruct((NC * NP, HALF), jnp.float32)] * K,
    scratch_types=[
        pltpu.VMEM_SHARED((NP, HALF), jnp.float32),
        pltpu.VMEM_SHARED((NP, HALF), jnp.float32),
        pltpu.VMEM((NBUF, 3, CHUNK), jnp.int32),
        pltpu.VMEM((NBUF, CHUNK, HALF), jnp.float32),
        pltpu.VMEM((ZR, HALF), jnp.float32),
        [pltpu.SemaphoreType.DMA] * NBUF,
        [pltpu.SemaphoreType.DMA] * NBUF,
        [pltpu.SemaphoreType.DMA] * NBUF,
    ],
    compiler_params=pltpu.CompilerParams(use_tc_tiling_on_sc=False,
                                         needs_layout_passes=False),
)(_sc_body)


BR = 1000
SGN = (1.0, -1.0, -1.0, 1.0)


def _tc_body(x_ref, t1_ref, t2_ref, t3_ref, th_ref, o_ref):
    acc = jnp.dot(x_ref[...], th_ref[0], preferred_element_type=jnp.float32)
    for k, tr in enumerate((t1_ref, t2_ref, t3_ref)):
        tcat = jnp.concatenate([tr[0], tr[1]], axis=1)
        prod = jnp.dot(tcat, th_ref[k + 1], preferred_element_type=jnp.float32)
        acc = acc + SGN[k + 1] * prod
    o_ref[...] = acc


def _tc_proj(x, t1, t2, t3, theta):
    tspec = pl.BlockSpec((2, BR, HALF), lambda i: (0, i, 0))
    return pl.pallas_call(
        _tc_body,
        grid=(N // BR,),
        in_specs=[
            pl.BlockSpec((BR, D), lambda i: (i, 0)),
            tspec, tspec, tspec,
            pl.BlockSpec((K + 1, D, D), lambda i: (0, 0, 0)),
        ],
        out_specs=pl.BlockSpec((BR, D), lambda i: (i, 0)),
        out_shape=jax.ShapeDtypeStruct((N, D), jnp.float32),
    )(x, t1, t2, t3, theta)


def kernel(x, edge_index, edge_vals, theta):
    rows = edge_index[0]
    cols = edge_index[1]
    pad = EP - E
    cols2d = jnp.pad(cols, (0, pad)).reshape(EPC, CHUNK)
    rows2d = jnp.pad(rows, (0, pad)).reshape(EPC, CHUNK)
    vp = jnp.pad(edge_vals, (0, pad))

    def asi32(a):
        return jax.lax.bitcast_convert_type(a, jnp.int32).reshape(EPC, CHUNK)

    pack = jnp.stack([cols2d, rows2d, asi32(vp)],
                     axis=1)
    rpad = NP - N
    x2 = jnp.concatenate([jnp.pad(x[:, :HALF], ((0, rpad), (0, 0))),
                          jnp.pad(x[:, HALF:], ((0, rpad), (0, 0)))], axis=0)
    t1, t2, t3 = _sc_spmv(x2, pack)
    return _tc_proj(x,
                    t1.reshape(NC, NP, HALF),
                    t2.reshape(NC, NP, HALF),
                    t3.reshape(NC, NP, HALF),
                    theta)
